# Initial kernel scaffold; baseline (speedup 1.0000x reference)
import jax
import jax.numpy as jnp
from jax import lax
from jax.experimental import pallas as pl
from jax.experimental.pallas import tpu as pltpu


def _test_kernel(x_ref, o_ref):
    v = x_ref[...]
    o_ref[...] = lax.slice(v, (0, 0), (8, 256), (1, 2))


def kernel(x, conv1_w, conv1_b, conv2_w, conv2_b, conv3_w, conv3_b,
           conv4_w, conv4_b, conv5_w, conv5_b, fc1_w, fc1_b, fc2_w, fc2_b):
    a = x.reshape(128, 3, -1)[0, 0, :2048].reshape(8, 256)
    r = pl.pallas_call(
        _test_kernel,
        out_shape=jax.ShapeDtypeStruct((8, 128), jnp.float32),
        grid=(1,),
        in_specs=[pl.BlockSpec((8, 256), lambda i: (0, 0))],
        out_specs=pl.BlockSpec((8, 128), lambda i: (0, 0)),
    )(a)
    return jnp.zeros((128, 36), jnp.float32) + r.sum()


# speedup vs baseline: 1.3460x; 1.3460x over previous
import functools

import numpy as np
import jax
import jax.numpy as jnp
from jax import lax
from jax.experimental import pallas as pl
from jax.experimental.pallas import tpu as pltpu

_LAYERS = (
    (3, 8, 140, 140),
    (8, 16, 68, 68),
    (16, 32, 32, 32),
    (32, 64, 14, 14),
)


def _chunked_idx(idx_np):
    chunks = []
    for j in range(-(-len(idx_np) // 128)):
        sub = idx_np[j * 128:(j + 1) * 128]
        if len(sub) < 128:
            sub = np.concatenate([sub, np.full(128 - len(sub), sub[-1])])
        chunks.append(sub)
    return chunks


def _lane_remap(v, idx_chunks, idx_ref, row_base):
    C, N = v.shape
    lane = lax.broadcasted_iota(jnp.int32, (C, 128), 1)
    chunks = []
    for j, sub in enumerate(idx_chunks):
        svreg = sub // 128
        gidx = jnp.broadcast_to(idx_ref[row_base + j:row_base + j + 1, :],
                                (C, 128))
        out = None
        for sv in np.unique(svreg):
            src = v[:, int(sv) * 128:int(sv) * 128 + 128]
            g = jnp.take_along_axis(src, gidx, axis=1)
            if out is None:
                out = g
            else:
                bnd = int(np.argmax(svreg == sv))
                out = jnp.where(lane < bnd, out, g)
        chunks.append(out)
    return jnp.concatenate(chunks, axis=1) if len(chunks) > 1 else chunks[0]


def _pool_idx(Hp, Wp, W):
    yp, xp = np.meshgrid(np.arange(Hp), np.arange(Wp), indexing="ij")
    return (2 * yp * W + 2 * xp).reshape(-1)


def _conv_layer(x_ref, w_ref, b_ref, xcol_ref, f_ref, store, idx_ref,
                row_base, *, Cin, Cout, H, W):
    Hout, Wout = H - 4, W - 4
    Hp, Wp = Hout // 2, Wout // 2
    S, Sc = H * W, Hout * W

    for dy in range(5):
        for dx in range(5):
            off = dy * W + dx
            ln = min(Sc, S - off)
            row = (dy * 5 + dx) * Cin
            xcol_ref[row:row + Cin, 0:ln] = x_ref[:, off:off + ln]

    f_ref[:, 0:Sc] = jnp.maximum(
        jnp.dot(w_ref[...], xcol_ref[...],
                preferred_element_type=jnp.float32) + b_ref[...], 0.0)

    v = f_ref[...]
    m = jnp.maximum(
        jnp.maximum(v[:, 0:Sc], v[:, 1:Sc + 1]),
        jnp.maximum(v[:, W:Sc + W], v[:, W + 1:Sc + W + 1]))
    res = _lane_remap(m, _chunked_idx(_pool_idx(Hp, Wp, W)), idx_ref,
                      row_base)
    store(res[:, :Hp * Wp])


def _convnet_kernel(x_ref, w1, b1, w2, b2, w3, b3, w4, b4, idx_ref, o_ref,
                    xcol1, f1, p1, xcol2, f2, p2, xcol3, f3, p3, xcol4, f4):
    ws = (w1, w2, w3, w4)
    bs = (b1, b2, b3, b4)
    xcols = (xcol1, xcol2, xcol3, xcol4)
    fs = (f1, f2, f3, f4)
    ps = (p1, p2, p3)

    src = x_ref
    row_base = 0
    for i, (cin, cout, h, w) in enumerate(_LAYERS):
        if i < 3:
            dst = ps[i]

            def store(val, dst=dst):
                dst[...] = val
        else:

            def store(val):
                o_ref[...] = val
        _conv_layer(src, ws[i], bs[i], xcols[i], fs[i], store, idx_ref,
                    row_base, Cin=cin, Cout=cout, H=h, W=w)
        hp, wp = (h - 4) // 2, (w - 4) // 2
        row_base += -(-(hp * wp) // 128)
        if i < 3:
            src = ps[i]


def _tail_kernel(x_ref, w5_ref, b5_ref, w6_ref, b6_ref, w7_ref, b7_ref,
                 o_ref):
    h = jnp.dot(x_ref[...], w5_ref[...], preferred_element_type=jnp.float32)
    h = jnp.maximum(h + b5_ref[...], 0.0)
    h = jnp.dot(h, w6_ref[...], preferred_element_type=jnp.float32)
    h = jnp.maximum(h + b6_ref[...], 0.0)
    o_ref[...] = (jnp.dot(h, w7_ref[...], preferred_element_type=jnp.float32)
                  + b7_ref[...])


def kernel(x, conv1_w, conv1_b, conv2_w, conv2_b, conv3_w, conv3_b,
           conv4_w, conv4_b, conv5_w, conv5_b, fc1_w, fc1_b, fc2_w, fc2_b):
    n = x.shape[0]
    xf = x.reshape(n, 3, -1)

    conv_ws = (conv1_w, conv2_w, conv3_w, conv4_w)
    conv_bs = (conv1_b, conv2_b, conv3_b, conv4_b)
    wmats, bcols = [], []
    for w, b in zip(conv_ws, conv_bs):
        cin, cout = w.shape[2], w.shape[3]
        wmats.append(jnp.transpose(w.reshape(25 * cin, cout)))
        bcols.append(b.reshape(cout, 1))

    scratch = []
    for (cin, cout, h, w) in _LAYERS:
        sc = (h - 4) * w
        scratch.append(pltpu.VMEM((25 * cin, sc), jnp.float32))
        scratch.append(pltpu.VMEM((cout, sc + w + 8), jnp.float32))
        hp, wp = (h - 4) // 2, (w - 4) // 2
        if h > 14:
            scratch.append(pltpu.VMEM((cout, hp * wp), jnp.float32))

    idx_rows = []
    for (cin, cout, h, w) in _LAYERS:
        hp, wp = (h - 4) // 2, (w - 4) // 2
        idx_rows.extend(c % 128 for c in _chunked_idx(_pool_idx(hp, wp, w)))
    idx_tab = jnp.asarray(np.stack(idx_rows).astype(np.int32))

    args = []
    in_specs = [pl.BlockSpec((None, 3, 140 * 140), lambda i: (i, 0, 0))]
    for wm, bc in zip(wmats, bcols):
        args.extend([wm, bc])
        in_specs.append(pl.BlockSpec(wm.shape, lambda i: (0, 0)))
        in_specs.append(pl.BlockSpec(bc.shape, lambda i: (0, 0)))
    args.append(idx_tab)
    in_specs.append(pl.BlockSpec(idx_tab.shape, lambda i: (0, 0)))

    p4 = pl.pallas_call(
        _convnet_kernel,
        out_shape=jax.ShapeDtypeStruct((n, 64, 25), jnp.float32),
        grid=(n,),
        in_specs=in_specs,
        out_specs=pl.BlockSpec((None, 64, 25), lambda i: (i, 0, 0)),
        scratch_shapes=scratch,
        compiler_params=pltpu.CompilerParams(
            dimension_semantics=("parallel",),
            vmem_limit_bytes=100 * 1024 * 1024),
    )(xf, *args)

    xflat = p4.reshape(n, 1600)
    w5 = jnp.transpose(conv5_w, (2, 0, 1, 3)).reshape(-1, conv5_w.shape[-1])

    targs = (xflat, w5, conv5_b.reshape(1, -1), fc1_w, fc1_b.reshape(1, -1),
             fc2_w, fc2_b.reshape(1, -1))
    nb = n // 2
    tspecs = [pl.BlockSpec((nb, 1600), lambda i: (i, 0))]
    tspecs += [pl.BlockSpec(a.shape, lambda i, nd=a.ndim: (0,) * nd)
               for a in targs[1:]]
    out = pl.pallas_call(
        _tail_kernel,
        out_shape=jax.ShapeDtypeStruct((n, fc2_w.shape[-1]), jnp.float32),
        grid=(2,),
        in_specs=tspecs,
        out_specs=pl.BlockSpec((nb, fc2_w.shape[-1]), lambda i: (i, 0)),
        compiler_params=pltpu.CompilerParams(
            dimension_semantics=("parallel",)),
    )(*targs)
    return out
